# baseline (speedup 1.0000x reference)
import jax
import jax.numpy as jnp
from jax.experimental import pallas as pl
from jax.experimental.pallas import tpu as pltpu


def _penalty_bcast_kernel(tok_ref, wphi_ref, out_ref):
    n_tok = tok_ref.shape[0]
    vblk = out_ref.shape[1]
    j = pl.program_id(1)
    base = j * vblk
    vids = jax.lax.broadcasted_iota(jnp.int32, (1, vblk), 1) + base
    acc = jnp.zeros((1, vblk), jnp.float32)
    for n in range(n_tok):
        acc = acc + jnp.where(vids == tok_ref[n], wphi_ref[n], 0.0)
    out_ref[:, :] = jnp.broadcast_to(acc, out_ref.shape)


def kernel(level, candidate_logits, tokens, phis, layer_weights):
    B, V = candidate_logits.shape
    N = tokens.shape[0]
    pool_tokens = tokens[:, level]
    w = jnp.cumsum(layer_weights)[level]
    wphi = phis * w

    BBLK = 256
    VBLK = 2048
    grid = (B // BBLK, pl.cdiv(V, VBLK))

    out = pl.pallas_call(
        _penalty_bcast_kernel,
        grid_spec=pltpu.PrefetchScalarGridSpec(
            num_scalar_prefetch=2,
            grid=grid,
            in_specs=[],
            out_specs=pl.BlockSpec((BBLK, VBLK), lambda b, v, *_: (b, v)),
        ),
        out_shape=jax.ShapeDtypeStruct((B, V), jnp.float32),
    )(pool_tokens, wphi)
    return out
